# baseline (speedup 1.0000x reference)
import functools

import jax
import jax.numpy as jnp
from jax import lax
from jax.experimental import pallas as pl
from jax.experimental.pallas import tpu as pltpu
from jax.experimental.pallas import tpu_sc as plsc

N = 10000
NPAD = 10240
E = 320000
D = 128
H = 256
HH = 128
G = 64
NS = 16
NC = 2
ROWS_PER_TILE = NPAD // NS

f32 = jnp.float32
i32 = jnp.int32


_EPW = E // (NS * NC)
_CB = 128
_NCH_D = _EPW // _CB
_REM_D = _EPW - _NCH_D * _CB

_sc_mesh = plsc.VectorSubcoreMesh(core_axis_name="c", subcore_axis_name="s")


@functools.partial(
    pl.kernel,
    out_type=jax.ShapeDtypeStruct((NC * NPAD,), f32),
    mesh=_sc_mesh,
    scratch_types=[
        pltpu.VMEM((_CB,), i32),
        pltpu.VMEM((_REM_D,), i32),
        pltpu.VMEM((_CB,), f32),
        pltpu.VMEM((_REM_D,), f32),
        pltpu.VMEM((ROWS_PER_TILE,), f32),
        pltpu.VMEM_SHARED((NPAD,), f32),
    ],
)
def _deg_kernel(dst_hbm, out_hbm, dv, dv16, ones_b, ones16, zb, deg_sh):
    c = lax.axis_index("c")
    s = lax.axis_index("s")
    w = s * NC + c
    for j in range(_CB // 16):
        ones_b[pl.ds(16 * j, 16)] = jnp.ones((16,), f32)
    ones16[...] = jnp.ones((_REM_D,), f32)
    for j in range(ROWS_PER_TILE // 16):
        zb[pl.ds(16 * j, 16)] = jnp.zeros((16,), f32)
    pltpu.sync_copy(zb, deg_sh.at[pl.ds(ROWS_PER_TILE * s, ROWS_PER_TILE)])
    plsc.subcore_barrier()

    def chunk(k, carry):
        base = pl.multiple_of(w * _EPW + k * _CB, 8)
        pltpu.sync_copy(dst_hbm.at[pl.ds(base, _CB)], dv)
        pltpu.sync_copy(ones_b, deg_sh.at[dv], add=True)
        return carry

    lax.fori_loop(0, _NCH_D, chunk, 0)
    base = pl.multiple_of(w * _EPW + _NCH_D * _CB, 8)
    pltpu.sync_copy(dst_hbm.at[pl.ds(base, _REM_D)], dv16)
    pltpu.sync_copy(ones16, deg_sh.at[dv16], add=True)
    plsc.subcore_barrier()
    pltpu.sync_copy(
        deg_sh.at[pl.ds(ROWS_PER_TILE * s, ROWS_PER_TILE)],
        out_hbm.at[pl.ds(c * NPAD + ROWS_PER_TILE * s, ROWS_PER_TILE)],
    )


_CPT = 160
_SEC = 16
_EPAD = NS * _CPT * _CB
_EROWS = _EPAD // _CB


@functools.partial(
    pl.kernel,
    out_type=(
        jax.ShapeDtypeStruct((NPAD, HH), f32),
        jax.ShapeDtypeStruct((NPAD, HH), f32),
    ),
    mesh=_sc_mesh,
    scratch_types=[
        pltpu.VMEM((_CB,), i32),
        pltpu.VMEM((_CB,), i32),
        pltpu.VMEM((_CB, HH), f32),
        pltpu.VMEM_SHARED((NPAD, HH), f32),
        pltpu.SemaphoreType.DMA,
    ],
)
def _msgpass_kernel(h0_hbm, h1_hbm, src_hbm, dst_hbm, out0_hbm, out1_hbm,
                    sv, dv, rows, acc_sh, sem):
    c = lax.axis_index("c")
    s = lax.axis_index("s")
    rsl = pl.ds(ROWS_PER_TILE * s, ROWS_PER_TILE)

    @pl.when(c == 0)
    def _():
        pltpu.sync_copy(h0_hbm.at[rsl], acc_sh.at[rsl])

    @pl.when(c == 1)
    def _():
        pltpu.sync_copy(h1_hbm.at[rsl], acc_sh.at[rsl])

    plsc.subcore_barrier()

    def edge_loop(h_hbm):
        def chunk(k, carry):
            base = pl.multiple_of(s * _CPT * _CB + k * _CB, 8)
            pltpu.sync_copy(src_hbm.at[pl.ds(base, _CB)], sv)
            pltpu.sync_copy(dst_hbm.at[pl.ds(base, _CB)], dv)
            pltpu.async_copy(h_hbm.at[sv], rows, sem).wait()
            pltpu.sync_copy(rows, acc_sh.at[dv], add=True)
            return carry

        lax.fori_loop(0, _CPT, chunk, 0)

    @pl.when(c == 0)
    def _():
        edge_loop(h0_hbm)

    @pl.when(c == 1)
    def _():
        edge_loop(h1_hbm)

    plsc.subcore_barrier()

    @pl.when(c == 0)
    def _():
        pltpu.sync_copy(acc_sh.at[rsl], out0_hbm.at[rsl])

    @pl.when(c == 1)
    def _():
        pltpu.sync_copy(acc_sh.at[rsl], out1_hbm.at[rsl])


def _graph_norm(t, batch_row, batch_col, gamma, beta, alpha, eps=1e-5):
    M = (lax.broadcasted_iota(i32, (G, NPAD), 0) == batch_row).astype(f32)
    MT = (lax.broadcasted_iota(i32, (NPAD, G), 1) == batch_col).astype(f32)
    cnt = jnp.maximum(jnp.sum(M, axis=1, keepdims=True), 1.0)
    mean = jnp.dot(M, t, preferred_element_type=f32) / cnt
    sub = t - alpha * jnp.dot(MT, mean, preferred_element_type=f32)
    var = jnp.dot(M, sub * sub, preferred_element_type=f32) / cnt
    rinv = lax.rsqrt(var + eps)
    return gamma * sub * jnp.dot(MT, rinv, preferred_element_type=f32) + beta


def _prelu(x, a):
    return jnp.where(x >= 0, x, a * x)


def _pre_body(x_ref, w1_ref, deg_ref, h0_ref, h1_ref, dinv_ref):
    h = jnp.dot(x_ref[...], w1_ref[...], preferred_element_type=f32)
    d2 = deg_ref[...]
    dinv = lax.rsqrt(d2[:, 0:1] + d2[:, 1:2] + 1.0)
    hs = h * dinv
    h0_ref[...] = hs[:, :HH]
    h1_ref[...] = hs[:, HH:]
    dinv_ref[...] = dinv


_pre_call = pl.pallas_call(
    _pre_body,
    out_shape=(
        jax.ShapeDtypeStruct((NPAD, HH), f32),
        jax.ShapeDtypeStruct((NPAD, HH), f32),
        jax.ShapeDtypeStruct((NPAD, 1), f32),
    ),
)


def _mid_body(a0_ref, a1_ref, dinv_ref, b_ref, gam_ref, bet_ref, al_ref,
              pr_ref, w2_ref, br_ref, bc_ref, o0_ref, o1_ref):
    acc = jnp.concatenate([a0_ref[...], a1_ref[...]], axis=1)
    dinv = dinv_ref[...]
    t = dinv * acc + b_ref[...]
    gn = _graph_norm(t, br_ref[...], bc_ref[...], gam_ref[...], bet_ref[...],
                     al_ref[...])
    hpos = _prelu(gn, pr_ref[...])
    hs2 = jnp.dot(hpos, w2_ref[...], preferred_element_type=f32) * dinv
    o0_ref[...] = hs2[:, :HH]
    o1_ref[...] = hs2[:, HH:]


_mid_call = pl.pallas_call(
    _mid_body,
    out_shape=(
        jax.ShapeDtypeStruct((NPAD, HH), f32),
        jax.ShapeDtypeStruct((NPAD, HH), f32),
    ),
)


def _post_body(a0_ref, a1_ref, dinv_ref, b_ref, gam_ref, bet_ref, al_ref,
               pr_ref, br_ref, bc_ref, wp1_ref, bp1_ref, pp_ref, wp2_ref,
               bp2_ref, out_ref, pooled_ref):
    acc = jnp.concatenate([a0_ref[...], a1_ref[...]], axis=1)
    t = dinv_ref[...] * acc + b_ref[...]
    gn = _graph_norm(t, br_ref[...], bc_ref[...], gam_ref[...], bet_ref[...],
                     al_ref[...])
    h2 = _prelu(gn, pr_ref[...])
    bc = bc_ref[...]

    def body(g, carry):
        m = jnp.max(jnp.where(bc == g, h2, -jnp.inf), axis=0, keepdims=True)
        pooled_ref[pl.ds(g, 1), :] = m
        return carry

    lax.fori_loop(0, G, body, 0)
    pooled = pooled_ref[...]
    z = jnp.dot(pooled, wp1_ref[...], preferred_element_type=f32) + bp1_ref[...]
    z = _prelu(z, pp_ref[...])
    z = jnp.dot(z, wp2_ref[...], preferred_element_type=f32) + bp2_ref[...]
    out_ref[...] = 1.0 / (1.0 + jnp.exp(-z))


_post_call = pl.pallas_call(
    _post_body,
    out_shape=jax.ShapeDtypeStruct((G, 1), f32),
    scratch_shapes=[pltpu.VMEM((G, H), f32)],
)


def kernel(x, edge_index, batch, W1, b1, gn1_gamma, gn1_beta, gn1_alpha,
           prelu1, W2, b2, gn2_gamma, gn2_beta, gn2_alpha, prelu2,
           Wp1, bp1, prelu_p, Wp2, bp2):
    src = edge_index[0]
    dst = edge_index[1]
    srcp = jnp.concatenate([src, jnp.full((_EPAD - E,), N, i32)])
    dstp = jnp.concatenate([dst, jnp.full((_EPAD - E,), NPAD - 1, i32)])
    xp = jnp.zeros((NPAD, D), f32).at[:N].set(x)
    batchp = jnp.concatenate([batch, jnp.full((NPAD - N,), G, i32)])
    batch_row = batchp.reshape(1, NPAD)
    batch_col = batchp.reshape(NPAD, 1)

    degflat = _deg_kernel(dst)
    degT = degflat.reshape(NC, NPAD).T

    h0, h1, dinv = _pre_call(xp, W1, degT)
    a0, a1 = _msgpass_kernel(h0, h1, srcp, dstp)
    hs0, hs1 = _mid_call(
        a0, a1, dinv, b1.reshape(1, H), gn1_gamma.reshape(1, H),
        gn1_beta.reshape(1, H), gn1_alpha.reshape(1, H), prelu1.reshape(1, H),
        W2, batch_row, batch_col)
    c0, c1 = _msgpass_kernel(hs0, hs1, srcp, dstp)
    out = _post_call(
        c0, c1, dinv, b2.reshape(1, H), gn2_gamma.reshape(1, H),
        gn2_beta.reshape(1, H), gn2_alpha.reshape(1, H), prelu2.reshape(1, H),
        batch_row, batch_col, Wp1, bp1.reshape(1, H), prelu_p.reshape(1, H),
        Wp2, bp2.reshape(1, 1))
    return out


# speedup vs baseline: 1.5457x; 1.5457x over previous
import functools

import jax
import jax.numpy as jnp
from jax import lax
from jax.experimental import pallas as pl
from jax.experimental.pallas import tpu as pltpu
from jax.experimental.pallas import tpu_sc as plsc

N = 10000
NPAD = 10240
E = 320000
D = 128
H = 256
HH = 128
G = 64
NS = 16
NC = 2
ROWS_PER_TILE = NPAD // NS

f32 = jnp.float32
i32 = jnp.int32


_EPW = E // (NS * NC)
_CB = 128
_NCH_D = _EPW // _CB
_REM_D = _EPW - _NCH_D * _CB

_sc_mesh = plsc.VectorSubcoreMesh(core_axis_name="c", subcore_axis_name="s")


@functools.partial(
    pl.kernel,
    out_type=jax.ShapeDtypeStruct((NC * NPAD,), f32),
    mesh=_sc_mesh,
    scratch_types=[
        pltpu.VMEM((_CB,), i32),
        pltpu.VMEM((_REM_D,), i32),
        pltpu.VMEM((_CB,), f32),
        pltpu.VMEM((_REM_D,), f32),
        pltpu.VMEM((ROWS_PER_TILE,), f32),
        pltpu.VMEM_SHARED((NPAD,), f32),
    ],
)
def _deg_kernel(dst_hbm, out_hbm, dv, dv16, ones_b, ones16, zb, deg_sh):
    c = lax.axis_index("c")
    s = lax.axis_index("s")
    w = s * NC + c
    for j in range(_CB // 16):
        ones_b[pl.ds(16 * j, 16)] = jnp.ones((16,), f32)
    ones16[...] = jnp.ones((_REM_D,), f32)
    for j in range(ROWS_PER_TILE // 16):
        zb[pl.ds(16 * j, 16)] = jnp.zeros((16,), f32)
    pltpu.sync_copy(zb, deg_sh.at[pl.ds(ROWS_PER_TILE * s, ROWS_PER_TILE)])
    plsc.subcore_barrier()

    def chunk(k, carry):
        base = pl.multiple_of(w * _EPW + k * _CB, 8)
        pltpu.sync_copy(dst_hbm.at[pl.ds(base, _CB)], dv)
        pltpu.sync_copy(ones_b, deg_sh.at[dv], add=True)
        return carry

    lax.fori_loop(0, _NCH_D, chunk, 0)
    base = pl.multiple_of(w * _EPW + _NCH_D * _CB, 8)
    pltpu.sync_copy(dst_hbm.at[pl.ds(base, _REM_D)], dv16)
    pltpu.sync_copy(ones16, deg_sh.at[dv16], add=True)
    plsc.subcore_barrier()
    pltpu.sync_copy(
        deg_sh.at[pl.ds(ROWS_PER_TILE * s, ROWS_PER_TILE)],
        out_hbm.at[pl.ds(c * NPAD + ROWS_PER_TILE * s, ROWS_PER_TILE)],
    )


_EPT = E // NS
_NCH_M = _EPT // _CB
_REM_M = _EPT - _NCH_M * _CB


@functools.partial(
    pl.kernel,
    out_type=(
        jax.ShapeDtypeStruct((NPAD, HH), f32),
        jax.ShapeDtypeStruct((NPAD, HH), f32),
    ),
    mesh=_sc_mesh,
    scratch_types=[
        pltpu.VMEM((_CB,), i32),
        pltpu.VMEM((_CB,), i32),
        pltpu.VMEM((_REM_M,), i32),
        pltpu.VMEM((_REM_M,), i32),
        pltpu.VMEM((_CB, HH), f32),
        pltpu.VMEM_SHARED((NPAD, HH), f32),
        pltpu.SemaphoreType.DMA,
    ],
)
def _msgpass_kernel(h0_hbm, h1_hbm, src_hbm, dst_hbm, out0_hbm, out1_hbm,
                    sv, dv, svr, dvr, rows, acc_sh, sem):
    c = lax.axis_index("c")
    s = lax.axis_index("s")
    rsl = pl.ds(ROWS_PER_TILE * s, ROWS_PER_TILE)

    @pl.when(c == 0)
    def _():
        pltpu.sync_copy(h0_hbm.at[rsl], acc_sh.at[rsl])

    @pl.when(c == 1)
    def _():
        pltpu.sync_copy(h1_hbm.at[rsl], acc_sh.at[rsl])

    plsc.subcore_barrier()

    def edge_loop(h_hbm):
        def chunk(k, carry):
            base = pl.multiple_of(s * _EPT + k * _CB, 8)
            pltpu.sync_copy(src_hbm.at[pl.ds(base, _CB)], sv)
            pltpu.sync_copy(dst_hbm.at[pl.ds(base, _CB)], dv)
            pltpu.async_copy(h_hbm.at[sv], rows, sem).wait()
            pltpu.sync_copy(rows, acc_sh.at[dv], add=True)
            return carry

        lax.fori_loop(0, _NCH_M, chunk, 0)
        base = pl.multiple_of(s * _EPT + _NCH_M * _CB, 8)
        pltpu.sync_copy(src_hbm.at[pl.ds(base, _REM_M)], svr)
        pltpu.sync_copy(dst_hbm.at[pl.ds(base, _REM_M)], dvr)
        pltpu.async_copy(h_hbm.at[svr], rows.at[pl.ds(0, _REM_M)], sem).wait()
        pltpu.sync_copy(rows.at[pl.ds(0, _REM_M)], acc_sh.at[dvr], add=True)

    @pl.when(c == 0)
    def _():
        edge_loop(h0_hbm)

    @pl.when(c == 1)
    def _():
        edge_loop(h1_hbm)

    plsc.subcore_barrier()

    @pl.when(c == 0)
    def _():
        pltpu.sync_copy(acc_sh.at[rsl], out0_hbm.at[rsl])

    @pl.when(c == 1)
    def _():
        pltpu.sync_copy(acc_sh.at[rsl], out1_hbm.at[rsl])


def _graph_norm(t, batch_row, batch_col, gamma, beta, alpha, eps=1e-5):
    M = (lax.broadcasted_iota(i32, (G, NPAD), 0) == batch_row).astype(f32)
    MT = (lax.broadcasted_iota(i32, (NPAD, G), 1) == batch_col).astype(f32)
    cnt = jnp.maximum(jnp.sum(M, axis=1, keepdims=True), 1.0)
    mean = jnp.dot(M, t, preferred_element_type=f32) / cnt
    sub = t - alpha * jnp.dot(MT, mean, preferred_element_type=f32)
    var = jnp.dot(M, sub * sub, preferred_element_type=f32) / cnt
    rinv = lax.rsqrt(var + eps)
    return gamma * sub * jnp.dot(MT, rinv, preferred_element_type=f32) + beta


def _prelu(x, a):
    return jnp.where(x >= 0, x, a * x)


def _pre_body(x_ref, w1_ref, deg_ref, h0_ref, h1_ref, dinv_ref):
    h = jnp.dot(x_ref[...], w1_ref[...], preferred_element_type=f32)
    d2 = deg_ref[...]
    dinv = lax.rsqrt(d2[:, 0:1] + d2[:, 1:2] + 1.0)
    hs = h * dinv
    h0_ref[...] = hs[:, :HH]
    h1_ref[...] = hs[:, HH:]
    dinv_ref[...] = dinv


_pre_call = pl.pallas_call(
    _pre_body,
    out_shape=(
        jax.ShapeDtypeStruct((NPAD, HH), f32),
        jax.ShapeDtypeStruct((NPAD, HH), f32),
        jax.ShapeDtypeStruct((NPAD, 1), f32),
    ),
)


def _mid_body(a0_ref, a1_ref, dinv_ref, b_ref, gam_ref, bet_ref, al_ref,
              pr_ref, w2_ref, br_ref, bc_ref, o0_ref, o1_ref):
    acc = jnp.concatenate([a0_ref[...], a1_ref[...]], axis=1)
    dinv = dinv_ref[...]
    t = dinv * acc + b_ref[...]
    gn = _graph_norm(t, br_ref[...], bc_ref[...], gam_ref[...], bet_ref[...],
                     al_ref[...])
    hpos = _prelu(gn, pr_ref[...])
    hs2 = jnp.dot(hpos, w2_ref[...], preferred_element_type=f32) * dinv
    o0_ref[...] = hs2[:, :HH]
    o1_ref[...] = hs2[:, HH:]


_mid_call = pl.pallas_call(
    _mid_body,
    out_shape=(
        jax.ShapeDtypeStruct((NPAD, HH), f32),
        jax.ShapeDtypeStruct((NPAD, HH), f32),
    ),
)


def _post_body(a0_ref, a1_ref, dinv_ref, b_ref, gam_ref, bet_ref, al_ref,
               pr_ref, br_ref, bc_ref, wp1_ref, bp1_ref, pp_ref, wp2_ref,
               bp2_ref, out_ref, pooled_ref):
    acc = jnp.concatenate([a0_ref[...], a1_ref[...]], axis=1)
    t = dinv_ref[...] * acc + b_ref[...]
    gn = _graph_norm(t, br_ref[...], bc_ref[...], gam_ref[...], bet_ref[...],
                     al_ref[...])
    h2 = _prelu(gn, pr_ref[...])
    bc = bc_ref[...]

    def body(g, carry):
        m = jnp.max(jnp.where(bc == g, h2, -jnp.inf), axis=0, keepdims=True)
        pooled_ref[pl.ds(g, 1), :] = m
        return carry

    lax.fori_loop(0, G, body, 0)
    pooled = pooled_ref[...]
    z = jnp.dot(pooled, wp1_ref[...], preferred_element_type=f32) + bp1_ref[...]
    z = _prelu(z, pp_ref[...])
    z = jnp.dot(z, wp2_ref[...], preferred_element_type=f32) + bp2_ref[...]
    out_ref[...] = 1.0 / (1.0 + jnp.exp(-z))


_post_call = pl.pallas_call(
    _post_body,
    out_shape=jax.ShapeDtypeStruct((G, 1), f32),
    scratch_shapes=[pltpu.VMEM((G, H), f32)],
)


def kernel(x, edge_index, batch, W1, b1, gn1_gamma, gn1_beta, gn1_alpha,
           prelu1, W2, b2, gn2_gamma, gn2_beta, gn2_alpha, prelu2,
           Wp1, bp1, prelu_p, Wp2, bp2):
    src = edge_index[0]
    dst = edge_index[1]
    xp = jnp.zeros((NPAD, D), f32).at[:N].set(x)
    batchp = jnp.concatenate([batch, jnp.full((NPAD - N,), G, i32)])
    batch_row = batchp.reshape(1, NPAD)
    batch_col = batchp.reshape(NPAD, 1)

    degflat = _deg_kernel(dst)
    degT = degflat.reshape(NC, NPAD).T

    h0, h1, dinv = _pre_call(xp, W1, degT)
    a0, a1 = _msgpass_kernel(h0, h1, src, dst)
    hs0, hs1 = _mid_call(
        a0, a1, dinv, b1.reshape(1, H), gn1_gamma.reshape(1, H),
        gn1_beta.reshape(1, H), gn1_alpha.reshape(1, H), prelu1.reshape(1, H),
        W2, batch_row, batch_col)
    c0, c1 = _msgpass_kernel(hs0, hs1, src, dst)
    out = _post_call(
        c0, c1, dinv, b2.reshape(1, H), gn2_gamma.reshape(1, H),
        gn2_beta.reshape(1, H), gn2_alpha.reshape(1, H), prelu2.reshape(1, H),
        batch_row, batch_col, Wp1, bp1.reshape(1, H), prelu_p.reshape(1, H),
        Wp2, bp2.reshape(1, 1))
    return out


# speedup vs baseline: 1.5740x; 1.0183x over previous
import functools

import jax
import jax.numpy as jnp
from jax import lax
from jax.experimental import pallas as pl
from jax.experimental.pallas import tpu as pltpu
from jax.experimental.pallas import tpu_sc as plsc

N = 10000
NPAD = 10240
E = 320000
D = 128
H = 256
HH = 128
G = 64
NS = 16
NC = 2
ROWS_PER_TILE = NPAD // NS

f32 = jnp.float32
i32 = jnp.int32


_EPW = E // (NS * NC)
_CB = 128
_NCH_D = _EPW // _CB
_REM_D = _EPW - _NCH_D * _CB

_sc_mesh = plsc.VectorSubcoreMesh(core_axis_name="c", subcore_axis_name="s")


@functools.partial(
    pl.kernel,
    out_type=jax.ShapeDtypeStruct((NC * NPAD,), f32),
    mesh=_sc_mesh,
    scratch_types=[
        pltpu.VMEM((_CB,), i32),
        pltpu.VMEM((_CB,), i32),
        pltpu.VMEM((_REM_D,), i32),
        pltpu.VMEM((_CB,), f32),
        pltpu.VMEM((_REM_D,), f32),
        pltpu.VMEM((ROWS_PER_TILE,), f32),
        pltpu.VMEM_SHARED((NPAD,), f32),
        pltpu.SemaphoreType.DMA,
        pltpu.SemaphoreType.DMA,
    ],
)
def _deg_kernel(dst_hbm, out_hbm, dva, dvb, dv16, ones_b, ones16, zb, deg_sh,
                sema, semb):
    c = lax.axis_index("c")
    s = lax.axis_index("s")
    w = s * NC + c
    for j in range(_CB // 16):
        ones_b[pl.ds(16 * j, 16)] = jnp.ones((16,), f32)
    ones16[...] = jnp.ones((_REM_D,), f32)
    for j in range(ROWS_PER_TILE // 16):
        zb[pl.ds(16 * j, 16)] = jnp.zeros((16,), f32)
    pltpu.sync_copy(zb, deg_sh.at[pl.ds(ROWS_PER_TILE * s, ROWS_PER_TILE)])
    plsc.subcore_barrier()

    dvs = (dva, dvb)
    sems = (sema, semb)

    def load(k, b):
        base = pl.multiple_of(w * _EPW + k * _CB, 8)
        pltpu.async_copy(dst_hbm.at[pl.ds(base, _CB)], dvs[b], sems[b])

    def drain(b):
        pltpu.make_async_copy(dst_hbm.at[pl.ds(0, _CB)], dvs[b],
                              sems[b]).wait()

    load(0, 0)

    def pair(p, carry):
        k = 2 * p
        load(k + 1, 1)
        drain(0)
        pltpu.sync_copy(ones_b, deg_sh.at[dva], add=True)

        @pl.when(k + 2 < _NCH_D)
        def _():
            load(k + 2, 0)

        drain(1)
        pltpu.sync_copy(ones_b, deg_sh.at[dvb], add=True)
        return carry

    lax.fori_loop(0, _NCH_D // 2, pair, 0)
    base = pl.multiple_of(w * _EPW + _NCH_D * _CB, 8)
    pltpu.sync_copy(dst_hbm.at[pl.ds(base, _REM_D)], dv16)
    pltpu.sync_copy(ones16, deg_sh.at[dv16], add=True)
    plsc.subcore_barrier()
    pltpu.sync_copy(
        deg_sh.at[pl.ds(ROWS_PER_TILE * s, ROWS_PER_TILE)],
        out_hbm.at[pl.ds(c * NPAD + ROWS_PER_TILE * s, ROWS_PER_TILE)],
    )


_EPT = E // NS
_NCH_M = _EPT // _CB
_REM_M = _EPT - _NCH_M * _CB


@functools.partial(
    pl.kernel,
    out_type=(
        jax.ShapeDtypeStruct((NPAD, HH), f32),
        jax.ShapeDtypeStruct((NPAD, HH), f32),
    ),
    mesh=_sc_mesh,
    scratch_types=[
        pltpu.VMEM((_CB,), i32),
        pltpu.VMEM((_CB,), i32),
        pltpu.VMEM((_REM_M,), i32),
        pltpu.VMEM((_REM_M,), i32),
        pltpu.VMEM((_CB, HH), f32),
        pltpu.VMEM_SHARED((NPAD, HH), f32),
        pltpu.SemaphoreType.DMA,
    ],
)
def _msgpass_kernel(h0_hbm, h1_hbm, src_hbm, dst_hbm, out0_hbm, out1_hbm,
                    sv, dv, svr, dvr, rows, acc_sh, sem):
    c = lax.axis_index("c")
    s = lax.axis_index("s")
    rsl = pl.ds(ROWS_PER_TILE * s, ROWS_PER_TILE)

    @pl.when(c == 0)
    def _():
        pltpu.sync_copy(h0_hbm.at[rsl], acc_sh.at[rsl])

    @pl.when(c == 1)
    def _():
        pltpu.sync_copy(h1_hbm.at[rsl], acc_sh.at[rsl])

    plsc.subcore_barrier()

    def edge_loop(h_hbm):
        def chunk(k, carry):
            base = pl.multiple_of(s * _EPT + k * _CB, 8)
            pltpu.sync_copy(src_hbm.at[pl.ds(base, _CB)], sv)
            pltpu.sync_copy(dst_hbm.at[pl.ds(base, _CB)], dv)
            pltpu.async_copy(h_hbm.at[sv], rows, sem).wait()
            pltpu.sync_copy(rows, acc_sh.at[dv], add=True)
            return carry

        lax.fori_loop(0, _NCH_M, chunk, 0)
        base = pl.multiple_of(s * _EPT + _NCH_M * _CB, 8)
        pltpu.sync_copy(src_hbm.at[pl.ds(base, _REM_M)], svr)
        pltpu.sync_copy(dst_hbm.at[pl.ds(base, _REM_M)], dvr)
        pltpu.async_copy(h_hbm.at[svr], rows.at[pl.ds(0, _REM_M)], sem).wait()
        pltpu.sync_copy(rows.at[pl.ds(0, _REM_M)], acc_sh.at[dvr], add=True)

    @pl.when(c == 0)
    def _():
        edge_loop(h0_hbm)

    @pl.when(c == 1)
    def _():
        edge_loop(h1_hbm)

    plsc.subcore_barrier()

    @pl.when(c == 0)
    def _():
        pltpu.sync_copy(acc_sh.at[rsl], out0_hbm.at[rsl])

    @pl.when(c == 1)
    def _():
        pltpu.sync_copy(acc_sh.at[rsl], out1_hbm.at[rsl])


def _graph_norm(t, batch_row, batch_col, gamma, beta, alpha, eps=1e-5):
    M = (lax.broadcasted_iota(i32, (G, NPAD), 0) == batch_row).astype(f32)
    MT = (lax.broadcasted_iota(i32, (NPAD, G), 1) == batch_col).astype(f32)
    cnt = jnp.maximum(jnp.sum(M, axis=1, keepdims=True), 1.0)
    mean = jnp.dot(M, t, preferred_element_type=f32) / cnt
    sub = t - alpha * jnp.dot(MT, mean, preferred_element_type=f32)
    var = jnp.dot(M, sub * sub, preferred_element_type=f32) / cnt
    rinv = lax.rsqrt(var + eps)
    return gamma * sub * jnp.dot(MT, rinv, preferred_element_type=f32) + beta


def _prelu(x, a):
    return jnp.where(x >= 0, x, a * x)


def _pre_body(x_ref, w1_ref, deg_ref, h0_ref, h1_ref, dinv_ref):
    h = jnp.dot(x_ref[...], w1_ref[...], preferred_element_type=f32)
    h = jnp.concatenate([h, jnp.zeros((NPAD - N, H), f32)], axis=0)
    d2 = deg_ref[...]
    dinv = lax.rsqrt(d2[:, 0:1] + d2[:, 1:2] + 1.0)
    hs = h * dinv
    h0_ref[...] = hs[:, :HH]
    h1_ref[...] = hs[:, HH:]
    dinv_ref[...] = dinv


_pre_call = pl.pallas_call(
    _pre_body,
    out_shape=(
        jax.ShapeDtypeStruct((NPAD, HH), f32),
        jax.ShapeDtypeStruct((NPAD, HH), f32),
        jax.ShapeDtypeStruct((NPAD, 1), f32),
    ),
)


def _mid_body(a0_ref, a1_ref, dinv_ref, b_ref, gam_ref, bet_ref, al_ref,
              pr_ref, w2_ref, br_ref, bc_ref, o0_ref, o1_ref):
    acc = jnp.concatenate([a0_ref[...], a1_ref[...]], axis=1)
    dinv = dinv_ref[...]
    t = dinv * acc + b_ref[...]
    gn = _graph_norm(t, br_ref[...], bc_ref[...], gam_ref[...], bet_ref[...],
                     al_ref[...])
    hpos = _prelu(gn, pr_ref[...])
    hs2 = jnp.dot(hpos, w2_ref[...], preferred_element_type=f32) * dinv
    o0_ref[...] = hs2[:, :HH]
    o1_ref[...] = hs2[:, HH:]


_mid_call = pl.pallas_call(
    _mid_body,
    out_shape=(
        jax.ShapeDtypeStruct((NPAD, HH), f32),
        jax.ShapeDtypeStruct((NPAD, HH), f32),
    ),
)


def _post_body(a0_ref, a1_ref, dinv_ref, b_ref, gam_ref, bet_ref, al_ref,
               pr_ref, br_ref, bc_ref, wp1_ref, bp1_ref, pp_ref, wp2_ref,
               bp2_ref, out_ref, pooled_ref):
    acc = jnp.concatenate([a0_ref[...], a1_ref[...]], axis=1)
    t = dinv_ref[...] * acc + b_ref[...]
    gn = _graph_norm(t, br_ref[...], bc_ref[...], gam_ref[...], bet_ref[...],
                     al_ref[...])
    h2 = _prelu(gn, pr_ref[...])
    bc = bc_ref[...]

    def body(g, carry):
        m = jnp.max(jnp.where(bc == g, h2, -jnp.inf), axis=0, keepdims=True)
        pooled_ref[pl.ds(g, 1), :] = m
        return carry

    lax.fori_loop(0, G, body, 0)
    pooled = pooled_ref[...]
    z = jnp.dot(pooled, wp1_ref[...], preferred_element_type=f32) + bp1_ref[...]
    z = _prelu(z, pp_ref[...])
    z = jnp.dot(z, wp2_ref[...], preferred_element_type=f32) + bp2_ref[...]
    out_ref[...] = 1.0 / (1.0 + jnp.exp(-z))


_post_call = pl.pallas_call(
    _post_body,
    out_shape=jax.ShapeDtypeStruct((G, 1), f32),
    scratch_shapes=[pltpu.VMEM((G, H), f32)],
)


def kernel(x, edge_index, batch, W1, b1, gn1_gamma, gn1_beta, gn1_alpha,
           prelu1, W2, b2, gn2_gamma, gn2_beta, gn2_alpha, prelu2,
           Wp1, bp1, prelu_p, Wp2, bp2):
    src = edge_index[0]
    dst = edge_index[1]
    batchp = jnp.concatenate([batch, jnp.full((NPAD - N,), G, i32)])
    batch_row = batchp.reshape(1, NPAD)
    batch_col = batchp.reshape(NPAD, 1)

    degflat = _deg_kernel(dst)
    degT = degflat.reshape(NC, NPAD).T

    h0, h1, dinv = _pre_call(x, W1, degT)
    a0, a1 = _msgpass_kernel(h0, h1, src, dst)
    hs0, hs1 = _mid_call(
        a0, a1, dinv, b1.reshape(1, H), gn1_gamma.reshape(1, H),
        gn1_beta.reshape(1, H), gn1_alpha.reshape(1, H), prelu1.reshape(1, H),
        W2, batch_row, batch_col)
    c0, c1 = _msgpass_kernel(hs0, hs1, src, dst)
    out = _post_call(
        c0, c1, dinv, b2.reshape(1, H), gn2_gamma.reshape(1, H),
        gn2_beta.reshape(1, H), gn2_alpha.reshape(1, H), prelu2.reshape(1, H),
        batch_row, batch_col, Wp1, bp1.reshape(1, H), prelu_p.reshape(1, H),
        Wp2, bp2.reshape(1, 1))
    return out


# speedup vs baseline: 2.0725x; 1.3167x over previous
import functools

import jax
import jax.numpy as jnp
from jax import lax
from jax.experimental import pallas as pl
from jax.experimental.pallas import tpu as pltpu
from jax.experimental.pallas import tpu_sc as plsc

N = 10000
NPAD = 10240
E = 320000
D = 128
H = 256
HH = 128
G = 64
NS = 16
NC = 2
ROWS_PER_TILE = NPAD // NS

f32 = jnp.float32
i32 = jnp.int32


_EPW = E // (NS * NC)
_CB = 128
_NCH_D = _EPW // _CB
_REM_D = _EPW - _NCH_D * _CB

_sc_mesh = plsc.VectorSubcoreMesh(core_axis_name="c", subcore_axis_name="s")


@functools.partial(
    pl.kernel,
    out_type=jax.ShapeDtypeStruct((NC * NPAD,), f32),
    mesh=_sc_mesh,
    scratch_types=[
        pltpu.VMEM((_CB,), i32),
        pltpu.VMEM((_CB,), i32),
        pltpu.VMEM((_REM_D,), i32),
        pltpu.VMEM((_CB,), f32),
        pltpu.VMEM((_REM_D,), f32),
        pltpu.VMEM((ROWS_PER_TILE,), f32),
        pltpu.VMEM_SHARED((NPAD,), f32),
        pltpu.SemaphoreType.DMA,
        pltpu.SemaphoreType.DMA,
    ],
)
def _deg_kernel(dst_hbm, out_hbm, dva, dvb, dv16, ones_b, ones16, zb, deg_sh,
                sema, semb):
    c = lax.axis_index("c")
    s = lax.axis_index("s")
    w = s * NC + c
    for j in range(_CB // 16):
        ones_b[pl.ds(16 * j, 16)] = jnp.ones((16,), f32)
    ones16[...] = jnp.ones((_REM_D,), f32)
    for j in range(ROWS_PER_TILE // 16):
        zb[pl.ds(16 * j, 16)] = jnp.zeros((16,), f32)
    pltpu.sync_copy(zb, deg_sh.at[pl.ds(ROWS_PER_TILE * s, ROWS_PER_TILE)])
    plsc.subcore_barrier()

    dvs = (dva, dvb)
    sems = (sema, semb)

    def load(k, b):
        base = pl.multiple_of(w * _EPW + k * _CB, 8)
        pltpu.async_copy(dst_hbm.at[pl.ds(base, _CB)], dvs[b], sems[b])

    def drain(b):
        pltpu.make_async_copy(dst_hbm.at[pl.ds(0, _CB)], dvs[b],
                              sems[b]).wait()

    load(0, 0)

    def pair(p, carry):
        k = 2 * p
        load(k + 1, 1)
        drain(0)
        pltpu.sync_copy(ones_b, deg_sh.at[dva], add=True)

        @pl.when(k + 2 < _NCH_D)
        def _():
            load(k + 2, 0)

        drain(1)
        pltpu.sync_copy(ones_b, deg_sh.at[dvb], add=True)
        return carry

    lax.fori_loop(0, _NCH_D // 2, pair, 0)
    base = pl.multiple_of(w * _EPW + _NCH_D * _CB, 8)
    pltpu.sync_copy(dst_hbm.at[pl.ds(base, _REM_D)], dv16)
    pltpu.sync_copy(ones16, deg_sh.at[dv16], add=True)
    plsc.subcore_barrier()
    pltpu.sync_copy(
        deg_sh.at[pl.ds(ROWS_PER_TILE * s, ROWS_PER_TILE)],
        out_hbm.at[pl.ds(c * NPAD + ROWS_PER_TILE * s, ROWS_PER_TILE)],
    )


_EPT = E // NS
_NCH_M = _EPT // _CB
_REM_M = _EPT - _NCH_M * _CB


@functools.partial(
    pl.kernel,
    out_type=(
        jax.ShapeDtypeStruct((NPAD, HH), f32),
        jax.ShapeDtypeStruct((NPAD, HH), f32),
    ),
    mesh=_sc_mesh,
    scratch_types=[
        pltpu.VMEM((_CB,), i32),
        pltpu.VMEM((_CB,), i32),
        pltpu.VMEM((_CB,), i32),
        pltpu.VMEM((_CB,), i32),
        pltpu.VMEM((_REM_M,), i32),
        pltpu.VMEM((_REM_M,), i32),
        pltpu.VMEM((_CB, HH), f32),
        pltpu.VMEM_SHARED((NPAD, HH), f32),
        pltpu.SemaphoreType.DMA,
        pltpu.SemaphoreType.DMA,
        pltpu.SemaphoreType.DMA,
    ],
)
def _msgpass_kernel(h0_hbm, h1_hbm, src_hbm, dst_hbm, out0_hbm, out1_hbm,
                    sva, dva, svb, dvb, svr, dvr, rows, acc_sh, sem,
                    ia_sem, ib_sem):
    c = lax.axis_index("c")
    s = lax.axis_index("s")
    rsl = pl.ds(ROWS_PER_TILE * s, ROWS_PER_TILE)

    @pl.when(c == 0)
    def _():
        pltpu.sync_copy(h0_hbm.at[rsl], acc_sh.at[rsl])

    @pl.when(c == 1)
    def _():
        pltpu.sync_copy(h1_hbm.at[rsl], acc_sh.at[rsl])

    plsc.subcore_barrier()

    def edge_loop(h_hbm):
        svs = (sva, svb)
        dvs = (dva, dvb)
        isems = (ia_sem, ib_sem)

        def fire(k, b):
            base = pl.multiple_of(s * _EPT + k * _CB, 8)
            pltpu.async_copy(src_hbm.at[pl.ds(base, _CB)], svs[b], isems[b])
            pltpu.async_copy(dst_hbm.at[pl.ds(base, _CB)], dvs[b], isems[b])

        def drain(b):
            pltpu.make_async_copy(src_hbm.at[pl.ds(0, _CB)], svs[b],
                                  isems[b]).wait()
            pltpu.make_async_copy(src_hbm.at[pl.ds(0, _CB)], dvs[b],
                                  isems[b]).wait()

        def work(b):
            pltpu.async_copy(h_hbm.at[svs[b]], rows, sem).wait()
            pltpu.sync_copy(rows, acc_sh.at[dvs[b]], add=True)

        fire(0, 0)

        def pair(p, carry):
            k = 2 * p
            fire(k + 1, 1)
            drain(0)
            work(0)

            @pl.when(k + 2 < _NCH_M)
            def _():
                fire(k + 2, 0)

            drain(1)
            work(1)
            return carry

        lax.fori_loop(0, _NCH_M // 2, pair, 0)
        base = pl.multiple_of(s * _EPT + _NCH_M * _CB, 8)
        pltpu.sync_copy(src_hbm.at[pl.ds(base, _REM_M)], svr)
        pltpu.sync_copy(dst_hbm.at[pl.ds(base, _REM_M)], dvr)
        pltpu.async_copy(h_hbm.at[svr], rows.at[pl.ds(0, _REM_M)], sem).wait()
        pltpu.sync_copy(rows.at[pl.ds(0, _REM_M)], acc_sh.at[dvr], add=True)

    @pl.when(c == 0)
    def _():
        edge_loop(h0_hbm)

    @pl.when(c == 1)
    def _():
        edge_loop(h1_hbm)

    plsc.subcore_barrier()

    @pl.when(c == 0)
    def _():
        pltpu.sync_copy(acc_sh.at[rsl], out0_hbm.at[rsl])

    @pl.when(c == 1)
    def _():
        pltpu.sync_copy(acc_sh.at[rsl], out1_hbm.at[rsl])


def _graph_norm(t, batch_row, batch_col, gamma, beta, alpha, eps=1e-5):
    M = (lax.broadcasted_iota(i32, (G, NPAD), 0) == batch_row).astype(f32)
    MT = (lax.broadcasted_iota(i32, (NPAD, G), 1) == batch_col).astype(f32)
    cnt = jnp.maximum(jnp.sum(M, axis=1, keepdims=True), 1.0)
    mean = jnp.dot(M, t, preferred_element_type=f32) / cnt
    sub = t - alpha * jnp.dot(MT, mean, preferred_element_type=f32)
    var = jnp.dot(M, sub * sub, preferred_element_type=f32) / cnt
    rinv = lax.rsqrt(var + eps)
    return gamma * sub * jnp.dot(MT, rinv, preferred_element_type=f32) + beta


def _prelu(x, a):
    return jnp.where(x >= 0, x, a * x)


def _pre_body(x_ref, w1_ref, deg_ref, h0_ref, h1_ref, dinv_ref):
    h = jnp.dot(x_ref[...], w1_ref[...], preferred_element_type=f32)
    h = jnp.concatenate([h, jnp.zeros((NPAD - N, H), f32)], axis=0)
    d2 = deg_ref[...]
    dinv = lax.rsqrt(d2[:, 0:1] + d2[:, 1:2] + 1.0)
    hs = h * dinv
    h0_ref[...] = hs[:, :HH]
    h1_ref[...] = hs[:, HH:]
    dinv_ref[...] = dinv


_pre_call = pl.pallas_call(
    _pre_body,
    out_shape=(
        jax.ShapeDtypeStruct((NPAD, HH), f32),
        jax.ShapeDtypeStruct((NPAD, HH), f32),
        jax.ShapeDtypeStruct((NPAD, 1), f32),
    ),
)


def _mid_body(a0_ref, a1_ref, dinv_ref, b_ref, gam_ref, bet_ref, al_ref,
              pr_ref, w2_ref, br_ref, bc_ref, o0_ref, o1_ref):
    acc = jnp.concatenate([a0_ref[...], a1_ref[...]], axis=1)
    dinv = dinv_ref[...]
    t = dinv * acc + b_ref[...]
    gn = _graph_norm(t, br_ref[...], bc_ref[...], gam_ref[...], bet_ref[...],
                     al_ref[...])
    hpos = _prelu(gn, pr_ref[...])
    hs2 = jnp.dot(hpos, w2_ref[...], preferred_element_type=f32) * dinv
    o0_ref[...] = hs2[:, :HH]
    o1_ref[...] = hs2[:, HH:]


_mid_call = pl.pallas_call(
    _mid_body,
    out_shape=(
        jax.ShapeDtypeStruct((NPAD, HH), f32),
        jax.ShapeDtypeStruct((NPAD, HH), f32),
    ),
)


def _post_body(a0_ref, a1_ref, dinv_ref, b_ref, gam_ref, bet_ref, al_ref,
               pr_ref, br_ref, bc_ref, wp1_ref, bp1_ref, pp_ref, wp2_ref,
               bp2_ref, out_ref, pooled_ref):
    acc = jnp.concatenate([a0_ref[...], a1_ref[...]], axis=1)
    t = dinv_ref[...] * acc + b_ref[...]
    gn = _graph_norm(t, br_ref[...], bc_ref[...], gam_ref[...], bet_ref[...],
                     al_ref[...])
    h2 = _prelu(gn, pr_ref[...])
    bc = bc_ref[...]

    def body(g, carry):
        m = jnp.max(jnp.where(bc == g, h2, -jnp.inf), axis=0, keepdims=True)
        pooled_ref[pl.ds(g, 1), :] = m
        return carry

    lax.fori_loop(0, G, body, 0)
    pooled = pooled_ref[...]
    z = jnp.dot(pooled, wp1_ref[...], preferred_element_type=f32) + bp1_ref[...]
    z = _prelu(z, pp_ref[...])
    z = jnp.dot(z, wp2_ref[...], preferred_element_type=f32) + bp2_ref[...]
    out_ref[...] = 1.0 / (1.0 + jnp.exp(-z))


_post_call = pl.pallas_call(
    _post_body,
    out_shape=jax.ShapeDtypeStruct((G, 1), f32),
    scratch_shapes=[pltpu.VMEM((G, H), f32)],
)


def kernel(x, edge_index, batch, W1, b1, gn1_gamma, gn1_beta, gn1_alpha,
           prelu1, W2, b2, gn2_gamma, gn2_beta, gn2_alpha, prelu2,
           Wp1, bp1, prelu_p, Wp2, bp2):
    src = edge_index[0]
    dst = edge_index[1]
    batchp = jnp.concatenate([batch, jnp.full((NPAD - N,), G, i32)])
    batch_row = batchp.reshape(1, NPAD)
    batch_col = batchp.reshape(NPAD, 1)

    degflat = _deg_kernel(dst)
    degT = degflat.reshape(NC, NPAD).T

    h0, h1, dinv = _pre_call(x, W1, degT)
    a0, a1 = _msgpass_kernel(h0, h1, src, dst)
    hs0, hs1 = _mid_call(
        a0, a1, dinv, b1.reshape(1, H), gn1_gamma.reshape(1, H),
        gn1_beta.reshape(1, H), gn1_alpha.reshape(1, H), prelu1.reshape(1, H),
        W2, batch_row, batch_col)
    c0, c1 = _msgpass_kernel(hs0, hs1, src, dst)
    out = _post_call(
        c0, c1, dinv, b2.reshape(1, H), gn2_gamma.reshape(1, H),
        gn2_beta.reshape(1, H), gn2_alpha.reshape(1, H), prelu2.reshape(1, H),
        batch_row, batch_col, Wp1, bp1.reshape(1, H), prelu_p.reshape(1, H),
        Wp2, bp2.reshape(1, 1))
    return out


# speedup vs baseline: 2.7390x; 1.3216x over previous
import functools

import jax
import jax.numpy as jnp
from jax import lax
from jax.experimental import pallas as pl
from jax.experimental.pallas import tpu as pltpu
from jax.experimental.pallas import tpu_sc as plsc

N = 10000
NPAD = 10240
E = 320000
D = 128
H = 256
HH = 128
G = 64
NS = 16
NC = 2
ROWS_PER_TILE = NPAD // NS

f32 = jnp.float32
i32 = jnp.int32


_EPW = E // (NS * NC)
_CB = 128
_NCH_D = _EPW // _CB
_REM_D = _EPW - _NCH_D * _CB

_sc_mesh = plsc.VectorSubcoreMesh(core_axis_name="c", subcore_axis_name="s")


@functools.partial(
    pl.kernel,
    out_type=jax.ShapeDtypeStruct((NC * NPAD,), f32),
    mesh=_sc_mesh,
    scratch_types=[
        pltpu.VMEM((_CB,), i32),
        pltpu.VMEM((_CB,), i32),
        pltpu.VMEM((_REM_D,), i32),
        pltpu.VMEM((_CB,), f32),
        pltpu.VMEM((_REM_D,), f32),
        pltpu.VMEM((ROWS_PER_TILE,), f32),
        pltpu.VMEM_SHARED((NPAD,), f32),
        pltpu.SemaphoreType.DMA,
        pltpu.SemaphoreType.DMA,
    ],
)
def _deg_kernel(dst_hbm, out_hbm, dva, dvb, dv16, ones_b, ones16, zb, deg_sh,
                sema, semb):
    c = lax.axis_index("c")
    s = lax.axis_index("s")
    w = s * NC + c
    for j in range(_CB // 16):
        ones_b[pl.ds(16 * j, 16)] = jnp.ones((16,), f32)
    ones16[...] = jnp.ones((_REM_D,), f32)
    for j in range(ROWS_PER_TILE // 16):
        zb[pl.ds(16 * j, 16)] = jnp.zeros((16,), f32)
    pltpu.sync_copy(zb, deg_sh.at[pl.ds(ROWS_PER_TILE * s, ROWS_PER_TILE)])
    plsc.subcore_barrier()

    dvs = (dva, dvb)
    sems = (sema, semb)

    def load(k, b):
        base = pl.multiple_of(w * _EPW + k * _CB, 8)
        pltpu.async_copy(dst_hbm.at[pl.ds(base, _CB)], dvs[b], sems[b])

    def drain(b):
        pltpu.make_async_copy(dst_hbm.at[pl.ds(0, _CB)], dvs[b],
                              sems[b]).wait()

    load(0, 0)

    def pair(p, carry):
        k = 2 * p
        load(k + 1, 1)
        drain(0)
        pltpu.sync_copy(ones_b, deg_sh.at[dva], add=True)

        @pl.when(k + 2 < _NCH_D)
        def _():
            load(k + 2, 0)

        drain(1)
        pltpu.sync_copy(ones_b, deg_sh.at[dvb], add=True)
        return carry

    lax.fori_loop(0, _NCH_D // 2, pair, 0)
    base = pl.multiple_of(w * _EPW + _NCH_D * _CB, 8)
    pltpu.sync_copy(dst_hbm.at[pl.ds(base, _REM_D)], dv16)
    pltpu.sync_copy(ones16, deg_sh.at[dv16], add=True)
    plsc.subcore_barrier()
    pltpu.sync_copy(
        deg_sh.at[pl.ds(ROWS_PER_TILE * s, ROWS_PER_TILE)],
        out_hbm.at[pl.ds(c * NPAD + ROWS_PER_TILE * s, ROWS_PER_TILE)],
    )


_EPT = E // NS
_NCH_M = _EPT // _CB
_REM_M = _EPT - _NCH_M * _CB


@functools.partial(
    pl.kernel,
    out_type=(
        jax.ShapeDtypeStruct((NPAD, HH), f32),
        jax.ShapeDtypeStruct((NPAD, HH), f32),
    ),
    mesh=_sc_mesh,
    scratch_types=[
        pltpu.VMEM((_CB,), i32),
        pltpu.VMEM((_CB,), i32),
        pltpu.VMEM((_CB,), i32),
        pltpu.VMEM((_CB,), i32),
        pltpu.VMEM((_REM_M,), i32),
        pltpu.VMEM((_REM_M,), i32),
        pltpu.VMEM((_CB, HH), f32),
        pltpu.VMEM((_CB, HH), f32),
        pltpu.VMEM_SHARED((NPAD, HH), f32),
        pltpu.SemaphoreType.DMA,
        pltpu.SemaphoreType.DMA,
        pltpu.SemaphoreType.DMA,
        pltpu.SemaphoreType.DMA,
    ],
)
def _msgpass_kernel(h0_hbm, h1_hbm, src_hbm, dst_hbm, out0_hbm, out1_hbm,
                    sva, dva, svb, dvb, svr, dvr, rows_a, rows_b, acc_sh,
                    ia_sem, ib_sem, ga_sem, gb_sem):
    c = lax.axis_index("c")
    s = lax.axis_index("s")
    rsl = pl.ds(ROWS_PER_TILE * s, ROWS_PER_TILE)

    @pl.when(c == 0)
    def _():
        pltpu.sync_copy(h0_hbm.at[rsl], acc_sh.at[rsl])

    @pl.when(c == 1)
    def _():
        pltpu.sync_copy(h1_hbm.at[rsl], acc_sh.at[rsl])

    plsc.subcore_barrier()

    def edge_loop(h_hbm):
        svs = (sva, svb)
        dvs = (dva, dvb)
        rows = (rows_a, rows_b)
        isems = (ia_sem, ib_sem)
        gsems = (ga_sem, gb_sem)

        def fire_idx(k, b):
            base = pl.multiple_of(s * _EPT + k * _CB, 8)
            pltpu.async_copy(src_hbm.at[pl.ds(base, _CB)], svs[b], isems[b])
            pltpu.async_copy(dst_hbm.at[pl.ds(base, _CB)], dvs[b], isems[b])

        def fire_gather(b):
            pltpu.make_async_copy(src_hbm.at[pl.ds(0, _CB)], svs[b],
                                  isems[b]).wait()
            pltpu.make_async_copy(src_hbm.at[pl.ds(0, _CB)], dvs[b],
                                  isems[b]).wait()
            pltpu.async_copy(h_hbm.at[svs[b]], rows[b], gsems[b])

        def scatter(b):
            pltpu.make_async_copy(h_hbm.at[pl.ds(0, _CB)], rows[b],
                                  gsems[b]).wait()
            pltpu.sync_copy(rows[b], acc_sh.at[dvs[b]], add=True)

        fire_idx(0, 0)
        fire_gather(0)
        fire_idx(1, 1)

        def pair(p, carry):
            k = 2 * p
            fire_gather(1)
            scatter(0)

            @pl.when(k + 2 < _NCH_M)
            def _():
                fire_idx(k + 2, 0)
                fire_gather(0)

            scatter(1)

            @pl.when(k + 3 < _NCH_M)
            def _():
                fire_idx(k + 3, 1)

            return carry

        lax.fori_loop(0, _NCH_M // 2, pair, 0)
        base = pl.multiple_of(s * _EPT + _NCH_M * _CB, 8)
        pltpu.sync_copy(src_hbm.at[pl.ds(base, _REM_M)], svr)
        pltpu.sync_copy(dst_hbm.at[pl.ds(base, _REM_M)], dvr)
        pltpu.async_copy(h_hbm.at[svr], rows_a.at[pl.ds(0, _REM_M)],
                         ga_sem).wait()
        pltpu.sync_copy(rows_a.at[pl.ds(0, _REM_M)], acc_sh.at[dvr], add=True)

    @pl.when(c == 0)
    def _():
        edge_loop(h0_hbm)

    @pl.when(c == 1)
    def _():
        edge_loop(h1_hbm)

    plsc.subcore_barrier()

    @pl.when(c == 0)
    def _():
        pltpu.sync_copy(acc_sh.at[rsl], out0_hbm.at[rsl])

    @pl.when(c == 1)
    def _():
        pltpu.sync_copy(acc_sh.at[rsl], out1_hbm.at[rsl])


def _graph_norm(t, batch_row, batch_col, gamma, beta, alpha, eps=1e-5):
    M = (lax.broadcasted_iota(i32, (G, NPAD), 0) == batch_row).astype(f32)
    MT = (lax.broadcasted_iota(i32, (NPAD, G), 1) == batch_col).astype(f32)
    cnt = jnp.maximum(jnp.sum(M, axis=1, keepdims=True), 1.0)
    mean = jnp.dot(M, t, preferred_element_type=f32) / cnt
    sub = t - alpha * jnp.dot(MT, mean, preferred_element_type=f32)
    var = jnp.dot(M, sub * sub, preferred_element_type=f32) / cnt
    rinv = lax.rsqrt(var + eps)
    return gamma * sub * jnp.dot(MT, rinv, preferred_element_type=f32) + beta


def _prelu(x, a):
    return jnp.where(x >= 0, x, a * x)


def _pre_body(x_ref, w1_ref, deg_ref, h0_ref, h1_ref, dinv_ref):
    h = jnp.dot(x_ref[...], w1_ref[...], preferred_element_type=f32)
    h = jnp.concatenate([h, jnp.zeros((NPAD - N, H), f32)], axis=0)
    d2 = deg_ref[...]
    dinv = lax.rsqrt(d2[:, 0:1] + d2[:, 1:2] + 1.0)
    hs = h * dinv
    h0_ref[...] = hs[:, :HH]
    h1_ref[...] = hs[:, HH:]
    dinv_ref[...] = dinv


_pre_call = pl.pallas_call(
    _pre_body,
    out_shape=(
        jax.ShapeDtypeStruct((NPAD, HH), f32),
        jax.ShapeDtypeStruct((NPAD, HH), f32),
        jax.ShapeDtypeStruct((NPAD, 1), f32),
    ),
)


def _mid_body(a0_ref, a1_ref, dinv_ref, b_ref, gam_ref, bet_ref, al_ref,
              pr_ref, w2_ref, br_ref, bc_ref, o0_ref, o1_ref):
    acc = jnp.concatenate([a0_ref[...], a1_ref[...]], axis=1)
    dinv = dinv_ref[...]
    t = dinv * acc + b_ref[...]
    gn = _graph_norm(t, br_ref[...], bc_ref[...], gam_ref[...], bet_ref[...],
                     al_ref[...])
    hpos = _prelu(gn, pr_ref[...])
    hs2 = jnp.dot(hpos, w2_ref[...], preferred_element_type=f32) * dinv
    o0_ref[...] = hs2[:, :HH]
    o1_ref[...] = hs2[:, HH:]


_mid_call = pl.pallas_call(
    _mid_body,
    out_shape=(
        jax.ShapeDtypeStruct((NPAD, HH), f32),
        jax.ShapeDtypeStruct((NPAD, HH), f32),
    ),
)


def _post_body(a0_ref, a1_ref, dinv_ref, b_ref, gam_ref, bet_ref, al_ref,
               pr_ref, br_ref, bc_ref, wp1_ref, bp1_ref, pp_ref, wp2_ref,
               bp2_ref, out_ref, pooled_ref):
    acc = jnp.concatenate([a0_ref[...], a1_ref[...]], axis=1)
    t = dinv_ref[...] * acc + b_ref[...]
    gn = _graph_norm(t, br_ref[...], bc_ref[...], gam_ref[...], bet_ref[...],
                     al_ref[...])
    h2 = _prelu(gn, pr_ref[...])
    bc = bc_ref[...]

    def body(g, carry):
        m = jnp.max(jnp.where(bc == g, h2, -jnp.inf), axis=0, keepdims=True)
        pooled_ref[pl.ds(g, 1), :] = m
        return carry

    lax.fori_loop(0, G, body, 0)
    pooled = pooled_ref[...]
    z = jnp.dot(pooled, wp1_ref[...], preferred_element_type=f32) + bp1_ref[...]
    z = _prelu(z, pp_ref[...])
    z = jnp.dot(z, wp2_ref[...], preferred_element_type=f32) + bp2_ref[...]
    out_ref[...] = 1.0 / (1.0 + jnp.exp(-z))


_post_call = pl.pallas_call(
    _post_body,
    out_shape=jax.ShapeDtypeStruct((G, 1), f32),
    scratch_shapes=[pltpu.VMEM((G, H), f32)],
)


def kernel(x, edge_index, batch, W1, b1, gn1_gamma, gn1_beta, gn1_alpha,
           prelu1, W2, b2, gn2_gamma, gn2_beta, gn2_alpha, prelu2,
           Wp1, bp1, prelu_p, Wp2, bp2):
    src = edge_index[0]
    dst = edge_index[1]
    batchp = jnp.concatenate([batch, jnp.full((NPAD - N,), G, i32)])
    batch_row = batchp.reshape(1, NPAD)
    batch_col = batchp.reshape(NPAD, 1)

    degflat = _deg_kernel(dst)
    degT = degflat.reshape(NC, NPAD).T

    h0, h1, dinv = _pre_call(x, W1, degT)
    a0, a1 = _msgpass_kernel(h0, h1, src, dst)
    hs0, hs1 = _mid_call(
        a0, a1, dinv, b1.reshape(1, H), gn1_gamma.reshape(1, H),
        gn1_beta.reshape(1, H), gn1_alpha.reshape(1, H), prelu1.reshape(1, H),
        W2, batch_row, batch_col)
    c0, c1 = _msgpass_kernel(hs0, hs1, src, dst)
    out = _post_call(
        c0, c1, dinv, b2.reshape(1, H), gn2_gamma.reshape(1, H),
        gn2_beta.reshape(1, H), gn2_alpha.reshape(1, H), prelu2.reshape(1, H),
        batch_row, batch_col, Wp1, bp1.reshape(1, H), prelu_p.reshape(1, H),
        Wp2, bp2.reshape(1, 1))
    return out
